# baseline (device time: 49605 ns/iter reference)
import jax
import jax.numpy as jnp
from jax import lax
from jax.experimental import pallas as pl
from jax.experimental.pallas import tpu as pltpu

N_DEV = 4
B, SQ, SKV, DH = 2, 256, 256, 64
HQ_TOTAL = 16
HQ_LOC = HQ_TOTAL // N_DEV
D_MODEL = 512
BLK = 64


def kernel(x, Wq, K_ext, V_ext, Wo):
    my = lax.axis_index("i")

    def pack(t):
        t = lax.dynamic_slice_in_dim(t, my * HQ_LOC, HQ_LOC, axis=2)
        t = t.transpose(0, 2, 1, 3).reshape(B * HQ_LOC, SKV, DH)
        return t.astype(jnp.bfloat16)

    k_loc = pack(K_ext)
    v_loc = pack(V_ext)
    x2 = x.reshape(B * SQ, D_MODEL).astype(jnp.bfloat16)
    wq = Wq.astype(jnp.bfloat16)
    wo = Wo.astype(jnp.bfloat16)

    def body(x_ref, wq_ref, k_ref, v_ref, wo_ref, out_ref,
             comm_ref, send_sems, recv_sems):
        my_pos = lax.axis_index("i")
        left = lax.rem(my_pos - 1 + N_DEV, N_DEV)
        right = lax.rem(my_pos + 1, N_DEV)

        barrier_sem = pltpu.get_barrier_semaphore()
        for nbr in (left, right):
            pl.semaphore_signal(
                barrier_sem, inc=1,
                device_id=(nbr,), device_id_type=pl.DeviceIdType.MESH,
            )
        pl.semaphore_wait(barrier_sem, 2)

        q = jnp.dot(x_ref[...], wq_ref[...],
                    preferred_element_type=jnp.float32)

        qb = lax.broadcasted_iota(jnp.int32, (SQ, SKV), 0) // BLK
        kb = lax.broadcasted_iota(jnp.int32, (SQ, SKV), 1) // BLK
        mask = kb <= qb

        ctx_parts = []
        for b in range(B):
            for h in range(HQ_LOC):
                qbh = q[b * SQ:(b + 1) * SQ, h * DH:(h + 1) * DH]
                qbh = qbh.astype(jnp.bfloat16)
                kbh = k_ref[b * HQ_LOC + h]
                s = lax.dot_general(
                    qbh, kbh, (((1,), (1,)), ((), ())),
                    preferred_element_type=jnp.float32,
                ) * 0.125
                s = jnp.where(mask, s, -1e9)
                m = jnp.max(s, axis=1, keepdims=True)
                w = jnp.exp(s - m)
                w = w / jnp.sum(w, axis=1, keepdims=True)
                ctx = jnp.dot(w.astype(jnp.bfloat16), v_ref[b * HQ_LOC + h],
                              preferred_element_type=jnp.float32)
                ctx_parts.append(ctx.astype(jnp.bfloat16))
        ctx_b = [jnp.concatenate(ctx_parts[b * HQ_LOC:(b + 1) * HQ_LOC], axis=1)
                 for b in range(B)]
        ctx_all = jnp.concatenate(ctx_b, axis=0)

        partial = jnp.dot(ctx_all, wo_ref[...],
                          preferred_element_type=jnp.float32)

        acc = partial
        comm_ref[0] = partial
        for h in range(N_DEV - 1):
            send_slot = h % 2
            recv_slot = (h + 1) % 2
            rdma = pltpu.make_async_remote_copy(
                src_ref=comm_ref.at[send_slot],
                dst_ref=comm_ref.at[recv_slot],
                send_sem=send_sems.at[send_slot],
                recv_sem=recv_sems.at[recv_slot],
                device_id=(right,),
                device_id_type=pl.DeviceIdType.MESH,
            )
            rdma.start()
            rdma.wait()
            acc = acc + comm_ref[recv_slot]

        out_ref[...] = acc.reshape(B, SQ, D_MODEL)

    return pl.pallas_call(
        body,
        out_shape=jax.ShapeDtypeStruct((B, SQ, D_MODEL), jnp.float32),
        in_specs=[pl.BlockSpec(memory_space=pltpu.VMEM)] * 5,
        out_specs=pl.BlockSpec(memory_space=pltpu.VMEM),
        scratch_shapes=[
            pltpu.VMEM((2, B * SQ, D_MODEL), jnp.float32),
            pltpu.SemaphoreType.DMA((2,)),
            pltpu.SemaphoreType.DMA((2,)),
        ],
        compiler_params=pltpu.CompilerParams(collective_id=0),
    )(x2, wq, k_loc, v_loc, wo)


# device time: 24954 ns/iter; 1.9879x vs baseline; 1.9879x over previous
import jax
import jax.numpy as jnp
from jax import lax
from jax.experimental import pallas as pl
from jax.experimental.pallas import tpu as pltpu

N_DEV = 4
B, SQ, SKV, DH = 2, 256, 256, 64
HQ_TOTAL = 16
HQ_LOC = HQ_TOTAL // N_DEV
D_MODEL = 512
BLK = 64


def kernel(x, Wq, K_ext, V_ext, Wo):
    my = lax.axis_index("i")

    def pack(t):
        t = lax.dynamic_slice_in_dim(t, my * HQ_LOC, HQ_LOC, axis=2)
        t = t.transpose(0, 2, 1, 3).reshape(B * HQ_LOC, SKV, DH)
        return t.astype(jnp.bfloat16)

    k_loc = pack(K_ext)
    v_loc = pack(V_ext)
    x2 = x.reshape(B * SQ, D_MODEL).astype(jnp.bfloat16)
    wq = Wq.astype(jnp.bfloat16)
    wo = Wo.astype(jnp.bfloat16)

    def body(x_ref, wq_ref, k_ref, v_ref, wo_ref, out_ref,
             comm_ref, send_ref, send_sems, recv_sems):
        my_pos = lax.axis_index("i")
        left = lax.rem(my_pos - 1 + N_DEV, N_DEV)
        right = lax.rem(my_pos + 1, N_DEV)

        barrier_sem = pltpu.get_barrier_semaphore()
        for nbr in (left, right):
            pl.semaphore_signal(
                barrier_sem, inc=1,
                device_id=(nbr,), device_id_type=pl.DeviceIdType.MESH,
            )
        pl.semaphore_wait(barrier_sem, 2)

        q = jnp.dot(x_ref[...], wq_ref[...],
                    preferred_element_type=jnp.float32)

        qb = lax.broadcasted_iota(jnp.int32, (SQ, SKV), 0) // BLK
        kb = lax.broadcasted_iota(jnp.int32, (SQ, SKV), 1) // BLK
        mask = kb <= qb

        ctx_parts = []
        for b in range(B):
            for h in range(HQ_LOC):
                qbh = q[b * SQ:(b + 1) * SQ, h * DH:(h + 1) * DH]
                qbh = qbh.astype(jnp.bfloat16)
                kbh = k_ref[b * HQ_LOC + h]
                s = lax.dot_general(
                    qbh, kbh, (((1,), (1,)), ((), ())),
                    preferred_element_type=jnp.float32,
                ) * 0.125
                s = jnp.where(mask, s, -1e9)
                m = jnp.max(s, axis=1, keepdims=True)
                w = jnp.exp(s - m)
                w = w / jnp.sum(w, axis=1, keepdims=True)
                ctx = jnp.dot(w.astype(jnp.bfloat16), v_ref[b * HQ_LOC + h],
                              preferred_element_type=jnp.float32)
                ctx_parts.append(ctx.astype(jnp.bfloat16))
        ctx_b = [jnp.concatenate(ctx_parts[b * HQ_LOC:(b + 1) * HQ_LOC], axis=1)
                 for b in range(B)]
        ctx_all = jnp.concatenate(ctx_b, axis=0)

        partial = jnp.dot(ctx_all, wo_ref[...],
                          preferred_element_type=jnp.float32)

        p1 = my_pos ^ 1
        p2 = (N_DEV - 1) - my_pos

        send_ref[0] = partial.astype(jnp.bfloat16)
        rdma1 = pltpu.make_async_remote_copy(
            src_ref=send_ref.at[0],
            dst_ref=comm_ref.at[0],
            send_sem=send_sems.at[0],
            recv_sem=recv_sems.at[0],
            device_id=(p1,),
            device_id_type=pl.DeviceIdType.MESH,
        )
        rdma1.start()
        rdma1.wait()
        acc = partial + comm_ref[0].astype(jnp.float32)

        send_ref[1] = acc.astype(jnp.bfloat16)
        rdma2 = pltpu.make_async_remote_copy(
            src_ref=send_ref.at[1],
            dst_ref=comm_ref.at[1],
            send_sem=send_sems.at[1],
            recv_sem=recv_sems.at[1],
            device_id=(p2,),
            device_id_type=pl.DeviceIdType.MESH,
        )
        rdma2.start()
        rdma2.wait()
        acc = acc + comm_ref[1].astype(jnp.float32)

        out_ref[...] = acc.reshape(B, SQ, D_MODEL)

    return pl.pallas_call(
        body,
        out_shape=jax.ShapeDtypeStruct((B, SQ, D_MODEL), jnp.float32),
        in_specs=[pl.BlockSpec(memory_space=pltpu.VMEM)] * 5,
        out_specs=pl.BlockSpec(memory_space=pltpu.VMEM),
        scratch_shapes=[
            pltpu.VMEM((2, B * SQ, D_MODEL), jnp.bfloat16),
            pltpu.VMEM((2, B * SQ, D_MODEL), jnp.bfloat16),
            pltpu.SemaphoreType.DMA((2,)),
            pltpu.SemaphoreType.DMA((2,)),
        ],
        compiler_params=pltpu.CompilerParams(collective_id=0),
    )(x2, wq, k_loc, v_loc, wo)


# device time: 20003 ns/iter; 2.4799x vs baseline; 1.2475x over previous
import jax
import jax.numpy as jnp
from jax import lax
from jax.experimental import pallas as pl
from jax.experimental.pallas import tpu as pltpu

N_DEV = 4
B, SQ, SKV, DH = 2, 256, 256, 64
HQ_TOTAL = 16
HQ_LOC = HQ_TOTAL // N_DEV
D_MODEL = 512
BLK = 64


def kernel(x, Wq, K_ext, V_ext, Wo):
    my = lax.axis_index("i")

    def pack(t):
        t = lax.dynamic_slice_in_dim(t, my * HQ_LOC, HQ_LOC, axis=2)
        t = t.astype(jnp.bfloat16)
        return t.transpose(0, 2, 1, 3).reshape(B * HQ_LOC, SKV, DH)

    k_loc = pack(K_ext)
    v_loc = pack(V_ext)
    x2 = x.reshape(B * SQ, D_MODEL).astype(jnp.bfloat16)
    wq = Wq.astype(jnp.bfloat16)
    wo = Wo.astype(jnp.bfloat16)

    def body(x_ref, wq_ref, k_ref, v_ref, wo_ref, out_ref,
             comm_ref, send_ref, send_sems, recv_sems):
        my_pos = lax.axis_index("i")
        p1 = my_pos ^ 1
        p2 = (N_DEV - 1) - my_pos

        barrier_sem = pltpu.get_barrier_semaphore()
        for nbr in (p1, p2):
            pl.semaphore_signal(
                barrier_sem, inc=1,
                device_id=(nbr,), device_id_type=pl.DeviceIdType.MESH,
            )
        pl.semaphore_wait(barrier_sem, 2)

        def exchange(slot, partner):
            return pltpu.make_async_remote_copy(
                src_ref=send_ref.at[slot],
                dst_ref=comm_ref.at[slot],
                send_sem=send_sems.at[slot],
                recv_sem=recv_sems.at[slot],
                device_id=(partner,),
                device_id_type=pl.DeviceIdType.MESH,
            )

        q = jnp.dot(x_ref[...], wq_ref[...],
                    preferred_element_type=jnp.float32)

        qb = lax.broadcasted_iota(jnp.int32, (SQ, SKV), 0) // BLK
        kb = lax.broadcasted_iota(jnp.int32, (SQ, SKV), 1) // BLK
        mask = kb <= qb

        def batch_partial(b):
            parts = []
            for h in range(HQ_LOC):
                qbh = q[b * SQ:(b + 1) * SQ, h * DH:(h + 1) * DH]
                qbh = qbh.astype(jnp.bfloat16)
                kbh = k_ref[b * HQ_LOC + h]
                s = lax.dot_general(
                    qbh, kbh, (((1,), (1,)), ((), ())),
                    preferred_element_type=jnp.float32,
                ) * 0.125
                s = jnp.where(mask, s, -1e9)
                m = jnp.max(s, axis=1, keepdims=True)
                w = jnp.exp(s - m)
                w = w / jnp.sum(w, axis=1, keepdims=True)
                ctx = jnp.dot(w.astype(jnp.bfloat16), v_ref[b * HQ_LOC + h],
                              preferred_element_type=jnp.float32)
                parts.append(ctx.astype(jnp.bfloat16))
            ctx_b = jnp.concatenate(parts, axis=1)
            return jnp.dot(ctx_b, wo_ref[...],
                           preferred_element_type=jnp.float32)

        pA = batch_partial(0)
        send_ref[0] = pA.astype(jnp.bfloat16)
        ex0 = exchange(0, p1)
        ex0.start()

        pB = batch_partial(1)
        send_ref[1] = pB.astype(jnp.bfloat16)
        ex1 = exchange(1, p2)
        ex1.start()

        ex0.wait()
        accA = pA + comm_ref[0].astype(jnp.float32)
        send_ref[2] = accA.astype(jnp.bfloat16)
        ex2 = exchange(2, p2)
        ex2.start()

        ex1.wait()
        accB = pB + comm_ref[1].astype(jnp.float32)
        send_ref[3] = accB.astype(jnp.bfloat16)
        ex3 = exchange(3, p1)
        ex3.start()

        ex2.wait()
        out_ref[0] = accA + comm_ref[2].astype(jnp.float32)
        ex3.wait()
        out_ref[1] = accB + comm_ref[3].astype(jnp.float32)

    return pl.pallas_call(
        body,
        out_shape=jax.ShapeDtypeStruct((B, SQ, D_MODEL), jnp.float32),
        in_specs=[pl.BlockSpec(memory_space=pltpu.VMEM)] * 5,
        out_specs=pl.BlockSpec(memory_space=pltpu.VMEM),
        scratch_shapes=[
            pltpu.VMEM((4, SQ, D_MODEL), jnp.bfloat16),
            pltpu.VMEM((4, SQ, D_MODEL), jnp.bfloat16),
            pltpu.SemaphoreType.DMA((4,)),
            pltpu.SemaphoreType.DMA((4,)),
        ],
        compiler_params=pltpu.CompilerParams(collective_id=0),
    )(x2, wq, k_loc, v_loc, wo)
